# baseline (speedup 1.0000x reference)
import jax
import jax.numpy as jnp
from jax.experimental import pallas as pl
from jax.experimental.pallas import tpu as pltpu

N_BLOCK = 1000


def _post_kernel(nf_ref, s_ref, m_ref, dirsum_ref, deg_ref, fdig_ref,
                 norm_ref, w_ref, b_ref, out_ref):
    nf = nf_ref[...]
    s = s_ref[...]
    m = m_ref[...]
    dirsum = dirsum_ref[...]
    deg = deg_ref[...]
    fdig = fdig_ref[...]
    norm = norm_ref[...]
    w = w_ref[...]
    b = b_ref[...]

    mean = s / jnp.maximum(deg, 1.0)
    maxv = jnp.where(m < -3e38, 0.0, m)
    dirv = dirsum - fdig * nf
    h = jnp.concatenate([nf, mean, maxv, dirv], axis=1)
    out = jnp.dot(h, w, preferred_element_type=jnp.float32) + b[0]
    out_ref[...] = nf + out * norm


def _post_transform(node_fts, s, m, dirsum, deg, fdig, norm_n, W_post, b_post):
    n, d = node_fts.shape
    grid = (n // N_BLOCK,)
    blk = lambda i: (i, 0)
    return pl.pallas_call(
        _post_kernel,
        grid=grid,
        in_specs=[
            pl.BlockSpec((N_BLOCK, d), blk),
            pl.BlockSpec((N_BLOCK, d), blk),
            pl.BlockSpec((N_BLOCK, d), blk),
            pl.BlockSpec((N_BLOCK, d), blk),
            pl.BlockSpec((N_BLOCK, 1), blk),
            pl.BlockSpec((N_BLOCK, 1), blk),
            pl.BlockSpec((N_BLOCK, 1), blk),
            pl.BlockSpec((4 * d, d), lambda i: (0, 0)),
            pl.BlockSpec((1, d), lambda i: (0, 0)),
        ],
        out_specs=pl.BlockSpec((N_BLOCK, d), blk),
        out_shape=jax.ShapeDtypeStruct((n, d), jnp.float32),
    )(node_fts, s, m, dirsum, deg, fdig, norm_n, W_post, b_post)


def kernel(node_fts, edge_fts, edge_index, F_norm_edge, F_dig, node_deg_vec,
           node_deg_mat, lap_mat, k_eig_val, k_eig_vec, num_nodes, norm_n,
           batch_idx, W_post, b_post):
    src = edge_index[0]
    dst = edge_index[1]
    n = node_fts.shape[0]
    x_src = jnp.take(node_fts, src, axis=0)
    s = jax.ops.segment_sum(x_src, dst, num_segments=n)
    m = jax.ops.segment_max(x_src, dst, num_segments=n)
    dirsum = jax.ops.segment_sum(F_norm_edge * x_src, dst, num_segments=n)
    return _post_transform(node_fts, s, m, dirsum, node_deg_vec, F_dig,
                           norm_n, W_post, b_post[None, :])


# speedup vs baseline: 1.5247x; 1.5247x over previous
import functools

import jax
import jax.numpy as jnp
from jax import lax
from jax.experimental import pallas as pl
from jax.experimental.pallas import tpu as pltpu
from jax.experimental.pallas import tpu_sc as plsc

N = 10000
D = 128
DH = 64
E = 320000
NW = 32
NPT = 320
NPAD = NW * NPT
NPS = 16 * NPT
CHUNK = 2000
NGRP = CHUNK // 16
K = 128
PCAP = 2176
NEG = float("-inf")


def _agg_body(nf2, src_h, dst_h, f_h, sum_o, max_o, dir_o,
              dchunk, schunk, fchunk, psrc, pdst, pf,
              idxbuf, dstlbuf, rows, scaled, maxacc, sum_sp, dir_sp, sem):
    c = lax.axis_index("c")
    s = lax.axis_index("s")
    tile_lo = (c * 16 + s) * NPT
    sc_base = s * NPT
    dummy_dst = tile_lo + NPT
    lane = lax.iota(jnp.int32, 16)

    def pass_body(h, carry):
        def init_max(i, cy):
            for j in range(DH // 16):
                maxacc[i, pl.ds(j * 16, 16)] = jnp.full((16,), NEG, jnp.float32)
            return cy
        lax.fori_loop(0, NPT + 8, init_max, 0)

        def zero_scaled(i, cy):
            for j in range(DH // 16):
                scaled[i, pl.ds(j * 16, 16)] = jnp.zeros((16,), jnp.float32)
            return cy
        lax.fori_loop(0, K, zero_scaled, 0)

        for acc in (sum_sp, dir_sp):
            pltpu.sync_copy(scaled, acc.at[pl.ds(sc_base, K)])
            pltpu.sync_copy(scaled, acc.at[pl.ds(sc_base + K, K)])
            pltpu.sync_copy(scaled.at[pl.ds(0, NPT - 2 * K)],
                            acc.at[pl.ds(sc_base + 2 * K, NPT - 2 * K)])

        def process_batch(base, nreal):
            for g in range(8):
                sv = psrc[pl.ds(base + g * 16, 16)]
                dv = pdst[pl.ds(base + g * 16, 16)]
                idxbuf[pl.ds(g * 16, 16)] = sv * 2 + h
                dstlbuf[pl.ds(g * 16, 16)] = dv - c * NPS
            pltpu.async_copy(nf2.at[idxbuf], rows, sem).wait()

            def zrow(e, cy):
                for j in range(DH // 16):
                    rows[e, pl.ds(j * 16, 16)] = jnp.zeros((16,), jnp.float32)
                return cy
            lax.fori_loop(nreal, K, zrow, 0)

            pltpu.sync_copy(rows, sum_sp.at[dstlbuf], add=True)

            def grp(g, cy):
                dv = pdst[pl.ds(base + g * 16, 16)]
                fv16 = pf[pl.ds(base + g * 16, 16)]
                for l in range(16):
                    dstl = jnp.max(jnp.where(lane == l, dv, 0)) - tile_lo
                    fsc = jnp.max(jnp.where(lane == l, fv16, NEG))
                    e = g * 16 + l
                    for j in range(DH // 16):
                        r = rows[e, pl.ds(j * 16, 16)]
                        scaled[e, pl.ds(j * 16, 16)] = r * fsc
                        a = maxacc[dstl, pl.ds(j * 16, 16)]
                        maxacc[dstl, pl.ds(j * 16, 16)] = jnp.maximum(a, r)
                return cy
            lax.fori_loop(0, 8, grp, 0)

            pltpu.sync_copy(scaled, dir_sp.at[dstlbuf], add=True)

        def scan_grp(g, p):
            d16 = dchunk[pl.ds(g * 16, 16)]
            msk = (d16 >= tile_lo) & (d16 < tile_lo + NPT)
            s16 = schunk[pl.ds(g * 16, 16)]
            f16 = fchunk[pl.ds(g * 16, 16)]
            mi = jnp.where(msk, 1, 0).astype(jnp.int32)
            cs = plsc.cumsum(mi)
            pos = cs - mi + p
            plsc.store_scatter(pdst, [pos], d16, mask=msk)
            plsc.store_scatter(psrc, [pos], s16, mask=msk)
            plsc.store_scatter(pf, [pos], f16, mask=msk)
            return p + jnp.max(cs)

        def drain_body(p):
            process_batch(p - K, K)
            return p - K

        def chunk_body(i, p):
            pltpu.sync_copy(dst_h.at[pl.ds(i * CHUNK, CHUNK)], dchunk)
            pltpu.sync_copy(src_h.at[pl.ds(i * CHUNK, CHUNK)], schunk)
            pltpu.sync_copy(f_h.at[pl.ds(i * CHUNK, CHUNK)], fchunk)
            p = lax.fori_loop(0, NGRP, scan_grp, p)
            p = lax.while_loop(lambda q: q >= K, drain_body, p)
            return p

        p = lax.fori_loop(0, E // CHUNK, chunk_body, jnp.int32(0))

        dummy_d = jnp.full((16,), 1, jnp.int32) * dummy_dst
        zi = jnp.zeros((16,), jnp.int32)
        zf = jnp.zeros((16,), jnp.float32)
        pdst[pl.ds(p, 16)] = dummy_d
        psrc[pl.ds(p, 16)] = zi
        pf[pl.ds(p, 16)] = zf
        for t in range(1, 8):
            @pl.when(t * 16 >= p)
            def _():
                pdst[pl.ds(t * 16, 16)] = dummy_d
                psrc[pl.ds(t * 16, 16)] = zi
                pf[pl.ds(t * 16, 16)] = zf

        @pl.when(p > 0)
        def _():
            process_batch(jnp.int32(0), p)

        pltpu.sync_copy(maxacc.at[pl.ds(0, NPT)],
                        max_o.at[h, pl.ds(tile_lo, NPT)])
        pltpu.sync_copy(sum_sp.at[pl.ds(sc_base, NPT)],
                        sum_o.at[h, pl.ds(tile_lo, NPT)])
        pltpu.sync_copy(dir_sp.at[pl.ds(sc_base, NPT)],
                        dir_o.at[h, pl.ds(tile_lo, NPT)])
        return carry

    lax.fori_loop(0, 2, pass_body, 0)


_agg = functools.partial(
    pl.kernel,
    mesh=plsc.VectorSubcoreMesh(core_axis_name="c", subcore_axis_name="s"),
    compiler_params=pltpu.CompilerParams(needs_layout_passes=False,
                                         use_tc_tiling_on_sc=False),
    out_type=[
        jax.ShapeDtypeStruct((2, NPAD, DH), jnp.float32),
        jax.ShapeDtypeStruct((2, NPAD, DH), jnp.float32),
        jax.ShapeDtypeStruct((2, NPAD, DH), jnp.float32),
    ],
    scratch_types=[
        pltpu.VMEM((CHUNK,), jnp.int32),
        pltpu.VMEM((CHUNK,), jnp.int32),
        pltpu.VMEM((CHUNK,), jnp.float32),
        pltpu.VMEM((PCAP,), jnp.int32),
        pltpu.VMEM((PCAP,), jnp.int32),
        pltpu.VMEM((PCAP,), jnp.float32),
        pltpu.VMEM((K,), jnp.int32),
        pltpu.VMEM((K,), jnp.int32),
        pltpu.VMEM((K, DH), jnp.float32),
        pltpu.VMEM((K, DH), jnp.float32),
        pltpu.VMEM((NPT + 8, DH), jnp.float32),
        pltpu.VMEM_SHARED((NPS + 8, DH), jnp.float32),
        pltpu.VMEM_SHARED((NPS + 8, DH), jnp.float32),
        pltpu.SemaphoreType.DMA,
    ],
)(_agg_body)


N_BLOCK = 1000


def _post_kernel(nf_ref, s_ref, m_ref, dirsum_ref, deg_ref, fdig_ref,
                 norm_ref, w_ref, b_ref, out_ref):
    nf = nf_ref[...]
    s = s_ref[...]
    m = m_ref[...]
    dirsum = dirsum_ref[...]
    deg = deg_ref[...]
    fdig = fdig_ref[...]
    norm = norm_ref[...]
    w = w_ref[...]
    b = b_ref[...]

    mean = s / jnp.maximum(deg, 1.0)
    maxv = jnp.where(jnp.isfinite(m), m, 0.0)
    dirv = dirsum - fdig * nf
    h = jnp.concatenate([nf, mean, maxv, dirv], axis=1)
    out = jnp.dot(h, w, preferred_element_type=jnp.float32) + b[0]
    out_ref[...] = nf + out * norm


def _post_transform(node_fts, s, m, dirsum, deg, fdig, norm_n, W_post, b_post):
    n, d = node_fts.shape
    grid = (n // N_BLOCK,)
    blk = lambda i: (i, 0)
    return pl.pallas_call(
        _post_kernel,
        grid=grid,
        in_specs=[
            pl.BlockSpec((N_BLOCK, d), blk),
            pl.BlockSpec((N_BLOCK, d), blk),
            pl.BlockSpec((N_BLOCK, d), blk),
            pl.BlockSpec((N_BLOCK, d), blk),
            pl.BlockSpec((N_BLOCK, 1), blk),
            pl.BlockSpec((N_BLOCK, 1), blk),
            pl.BlockSpec((N_BLOCK, 1), blk),
            pl.BlockSpec((4 * d, d), lambda i: (0, 0)),
            pl.BlockSpec((1, d), lambda i: (0, 0)),
        ],
        out_specs=pl.BlockSpec((N_BLOCK, d), blk),
        out_shape=jax.ShapeDtypeStruct((n, d), jnp.float32),
    )(node_fts, s, m, dirsum, deg, fdig, norm_n, W_post, b_post)


def kernel(node_fts, edge_fts, edge_index, F_norm_edge, F_dig, node_deg_vec,
           node_deg_mat, lap_mat, k_eig_val, k_eig_vec, num_nodes, norm_n,
           batch_idx, W_post, b_post):
    src = edge_index[0]
    dst = edge_index[1]
    f = F_norm_edge[:, 0]
    nf2 = node_fts.reshape(2 * N, DH)
    s3, m3, dir3 = _agg(nf2, src, dst, f)
    s = jnp.concatenate([s3[0, :N], s3[1, :N]], axis=1)
    m = jnp.concatenate([m3[0, :N], m3[1, :N]], axis=1)
    dirsum = jnp.concatenate([dir3[0, :N], dir3[1, :N]], axis=1)
    return _post_transform(node_fts, s, m, dirsum, node_deg_vec, F_dig,
                           norm_n, W_post, b_post[None, :])


# speedup vs baseline: 2.0359x; 1.3352x over previous
import functools

import jax
import jax.numpy as jnp
from jax import lax
from jax.experimental import pallas as pl
from jax.experimental.pallas import tpu as pltpu
from jax.experimental.pallas import tpu_sc as plsc

N = 10000
D = 128
DH = 64
E = 320000
NW = 32
NPT = 320
NPAD = NW * NPT
NPS = 16 * NPT
CHUNK = 1600
NGRP = CHUNK // 16
NCHUNK = E // CHUNK
K = 128
PCAP = 12160
NEG = float("-inf")


def _agg_body(nf2, src_h, dst_h, f_h, sum_o, max_o, dir_o,
              dchunk, schunk, fchunk, psrc, pdst, pf,
              idxbuf, dstlbuf, rows, scaled, maxacc, sum_sp, dir_sp,
              dsem, gsem):
    c = lax.axis_index("c")
    s = lax.axis_index("s")
    tile_lo = (c * 16 + s) * NPT
    sc_base = s * NPT
    dummy_dst = tile_lo + NPT
    lane = lax.iota(jnp.int32, 16)

    def pass_body(h, carry):
        def init_max(i, cy):
            for j in range(DH // 16):
                maxacc[i, pl.ds(j * 16, 16)] = jnp.full((16,), NEG, jnp.float32)
            return cy
        lax.fori_loop(0, NPT + 8, init_max, 0)

        def zero_scaled(i, cy):
            for j in range(DH // 16):
                scaled[i, pl.ds(j * 16, 16)] = jnp.zeros((16,), jnp.float32)
            return cy
        lax.fori_loop(0, K, zero_scaled, 0)

        for acc in (sum_sp, dir_sp):
            pltpu.sync_copy(scaled.at[pl.ds(0, K)], acc.at[pl.ds(sc_base, K)])
            pltpu.sync_copy(scaled.at[pl.ds(0, K)], acc.at[pl.ds(sc_base + K, K)])
            pltpu.sync_copy(scaled.at[pl.ds(0, NPT - 2 * K)],
                            acc.at[pl.ds(sc_base + 2 * K, NPT - 2 * K)])

        def process_batch(base, nreal):
            for g in range(8):
                sv = psrc[pl.ds(base + g * 16, 16)]
                dv = pdst[pl.ds(base + g * 16, 16)]
                idxbuf[pl.ds(g * 16, 16)] = sv * 2 + h
                dstlbuf[pl.ds(g * 16, 16)] = dv - c * NPS
            pltpu.async_copy(nf2.at[idxbuf], rows, gsem).wait()

            def zrow(e, cy):
                for j in range(DH // 16):
                    rows[e, pl.ds(j * 16, 16)] = jnp.zeros((16,), jnp.float32)
                return cy
            lax.fori_loop(nreal, K, zrow, 0)

            pltpu.sync_copy(rows, sum_sp.at[dstlbuf], add=True)

            def grp(g, cy):
                dv = pdst[pl.ds(base + g * 16, 16)]
                fv16 = pf[pl.ds(base + g * 16, 16)]
                for l in range(16):
                    dstl = jnp.max(jnp.where(lane == l, dv, 0)) - tile_lo
                    fsc = jnp.max(jnp.where(lane == l, fv16, NEG))
                    e = g * 16 + l
                    for j in range(DH // 16):
                        r = rows[e, pl.ds(j * 16, 16)]
                        scaled[e, pl.ds(j * 16, 16)] = r * fsc
                        a = maxacc[dstl, pl.ds(j * 16, 16)]
                        maxacc[dstl, pl.ds(j * 16, 16)] = jnp.maximum(a, r)
                return cy
            lax.fori_loop(0, 8, grp, 0)

            pltpu.sync_copy(scaled, dir_sp.at[dstlbuf], add=True)

        def scan_chunk(par, p):
            def scan_grp(g, q):
                d16 = dchunk[par, pl.ds(g * 16, 16)]
                msk = (d16 >= tile_lo) & (d16 < tile_lo + NPT)
                s16 = schunk[par, pl.ds(g * 16, 16)]
                f16 = fchunk[par, pl.ds(g * 16, 16)]
                mi = jnp.where(msk, 1, 0).astype(jnp.int32)
                cs = plsc.cumsum(mi)
                pos = cs - mi + q
                plsc.store_scatter(pdst, [pos], d16, mask=msk)
                plsc.store_scatter(psrc, [pos], s16, mask=msk)
                plsc.store_scatter(pf, [pos], f16, mask=msk)
                return q + jnp.max(cs)
            return lax.fori_loop(0, NGRP, scan_grp, p)

        def drain(p):
            nb = p // K

            def dbody(b, cy):
                process_batch(b * K, K)
                return cy
            lax.fori_loop(0, nb, dbody, 0)
            r = p - nb * K

            @pl.when(nb > 0)
            def _():
                def mv(g, cy):
                    sv = psrc[pl.ds(nb * K + g * 16, 16)]
                    dv = pdst[pl.ds(nb * K + g * 16, 16)]
                    fv = pf[pl.ds(nb * K + g * 16, 16)]
                    psrc[pl.ds(g * 16, 16)] = sv
                    pdst[pl.ds(g * 16, 16)] = dv
                    pf[pl.ds(g * 16, 16)] = fv
                    return cy
                lax.fori_loop(0, 8, mv, 0)
            return r

        def issue_chunk(i, par):
            pltpu.async_copy(dst_h.at[pl.ds(i * CHUNK, CHUNK)],
                             dchunk.at[par], dsem)
            pltpu.async_copy(src_h.at[pl.ds(i * CHUNK, CHUNK)],
                             schunk.at[par], dsem)
            pltpu.async_copy(f_h.at[pl.ds(i * CHUNK, CHUNK)],
                             fchunk.at[par], dsem)

        def wait_chunk(i, par):
            pltpu.make_async_copy(dst_h.at[pl.ds(i * CHUNK, CHUNK)],
                                  dchunk.at[par], dsem).wait()
            pltpu.make_async_copy(src_h.at[pl.ds(i * CHUNK, CHUNK)],
                                  schunk.at[par], dsem).wait()
            pltpu.make_async_copy(f_h.at[pl.ds(i * CHUNK, CHUNK)],
                                  fchunk.at[par], dsem).wait()

        def chunk_body(i, p):
            par = lax.rem(i, 2)
            wait_chunk(i, par)

            @pl.when(i + 1 < NCHUNK)
            def _():
                issue_chunk(i + 1, 1 - par)
            p = scan_chunk(par, p)
            p = lax.cond(p >= PCAP - CHUNK, drain, lambda q: q, p)
            return p

        issue_chunk(0, 0)
        p = lax.fori_loop(0, NCHUNK, chunk_body, jnp.int32(0))

        nb = p // K

        def bat(b, cy):
            process_batch(b * K, K)
            return cy
        lax.fori_loop(0, nb, bat, 0)
        r = p - nb * K
        tbase = nb * K

        @pl.when(r > 0)
        def _():
            dummy_d = jnp.full((16,), 1, jnp.int32) * dummy_dst
            zi = jnp.zeros((16,), jnp.int32)
            zf = jnp.zeros((16,), jnp.float32)
            pdst[pl.ds(p, 16)] = dummy_d
            psrc[pl.ds(p, 16)] = zi
            pf[pl.ds(p, 16)] = zf
            for t in range(1, 8):
                @pl.when(tbase + t * 16 >= p)
                def _():
                    pdst[pl.ds(tbase + t * 16, 16)] = dummy_d
                    psrc[pl.ds(tbase + t * 16, 16)] = zi
                    pf[pl.ds(tbase + t * 16, 16)] = zf
            process_batch(tbase, r)

        pltpu.sync_copy(maxacc.at[pl.ds(0, NPT)],
                        max_o.at[h, pl.ds(tile_lo, NPT)])
        pltpu.sync_copy(sum_sp.at[pl.ds(sc_base, NPT)],
                        sum_o.at[h, pl.ds(tile_lo, NPT)])
        pltpu.sync_copy(dir_sp.at[pl.ds(sc_base, NPT)],
                        dir_o.at[h, pl.ds(tile_lo, NPT)])
        return carry

    lax.fori_loop(0, 2, pass_body, 0)


_agg = functools.partial(
    pl.kernel,
    mesh=plsc.VectorSubcoreMesh(core_axis_name="c", subcore_axis_name="s"),
    compiler_params=pltpu.CompilerParams(needs_layout_passes=False,
                                         use_tc_tiling_on_sc=False),
    out_type=[
        jax.ShapeDtypeStruct((2, NPAD, DH), jnp.float32),
        jax.ShapeDtypeStruct((2, NPAD, DH), jnp.float32),
        jax.ShapeDtypeStruct((2, NPAD, DH), jnp.float32),
    ],
    scratch_types=[
        pltpu.VMEM((2, CHUNK), jnp.int32),
        pltpu.VMEM((2, CHUNK), jnp.int32),
        pltpu.VMEM((2, CHUNK), jnp.float32),
        pltpu.VMEM((PCAP + 128,), jnp.int32),
        pltpu.VMEM((PCAP + 128,), jnp.int32),
        pltpu.VMEM((PCAP + 128,), jnp.float32),
        pltpu.VMEM((K,), jnp.int32),
        pltpu.VMEM((K,), jnp.int32),
        pltpu.VMEM((K, DH), jnp.float32),
        pltpu.VMEM((K, DH), jnp.float32),
        pltpu.VMEM((NPT + 8, DH), jnp.float32),
        pltpu.VMEM_SHARED((NPS + 8, DH), jnp.float32),
        pltpu.VMEM_SHARED((NPS + 8, DH), jnp.float32),
        pltpu.SemaphoreType.DMA,
        pltpu.SemaphoreType.DMA,
    ],
)(_agg_body)


N_BLOCK = 1000


def _post_kernel(nf_ref, s_ref, m_ref, dirsum_ref, deg_ref, fdig_ref,
                 norm_ref, w_ref, b_ref, out_ref):
    nf = nf_ref[...]
    s = s_ref[...]
    m = m_ref[...]
    dirsum = dirsum_ref[...]
    deg = deg_ref[...]
    fdig = fdig_ref[...]
    norm = norm_ref[...]
    w = w_ref[...]
    b = b_ref[...]

    mean = s / jnp.maximum(deg, 1.0)
    maxv = jnp.where(jnp.isfinite(m), m, 0.0)
    dirv = dirsum - fdig * nf
    h = jnp.concatenate([nf, mean, maxv, dirv], axis=1)
    out = jnp.dot(h, w, preferred_element_type=jnp.float32) + b[0]
    out_ref[...] = nf + out * norm


def _post_transform(node_fts, s, m, dirsum, deg, fdig, norm_n, W_post, b_post):
    n, d = node_fts.shape
    grid = (n // N_BLOCK,)
    blk = lambda i: (i, 0)
    return pl.pallas_call(
        _post_kernel,
        grid=grid,
        in_specs=[
            pl.BlockSpec((N_BLOCK, d), blk),
            pl.BlockSpec((N_BLOCK, d), blk),
            pl.BlockSpec((N_BLOCK, d), blk),
            pl.BlockSpec((N_BLOCK, d), blk),
            pl.BlockSpec((N_BLOCK, 1), blk),
            pl.BlockSpec((N_BLOCK, 1), blk),
            pl.BlockSpec((N_BLOCK, 1), blk),
            pl.BlockSpec((4 * d, d), lambda i: (0, 0)),
            pl.BlockSpec((1, d), lambda i: (0, 0)),
        ],
        out_specs=pl.BlockSpec((N_BLOCK, d), blk),
        out_shape=jax.ShapeDtypeStruct((n, d), jnp.float32),
    )(node_fts, s, m, dirsum, deg, fdig, norm_n, W_post, b_post)


def kernel(node_fts, edge_fts, edge_index, F_norm_edge, F_dig, node_deg_vec,
           node_deg_mat, lap_mat, k_eig_val, k_eig_vec, num_nodes, norm_n,
           batch_idx, W_post, b_post):
    src = edge_index[0]
    dst = edge_index[1]
    f = F_norm_edge[:, 0]
    nf2 = node_fts.reshape(2 * N, DH)
    s3, m3, dir3 = _agg(nf2, src, dst, f)
    s = jnp.concatenate([s3[0, :N], s3[1, :N]], axis=1)
    m = jnp.concatenate([m3[0, :N], m3[1, :N]], axis=1)
    dirsum = jnp.concatenate([dir3[0, :N], dir3[1, :N]], axis=1)
    return _post_transform(node_fts, s, m, dirsum, node_deg_vec, F_dig,
                           norm_n, W_post, b_post[None, :])


# speedup vs baseline: 2.2945x; 1.1270x over previous
import functools

import jax
import jax.numpy as jnp
from jax import lax
from jax.experimental import pallas as pl
from jax.experimental.pallas import tpu as pltpu
from jax.experimental.pallas import tpu_sc as plsc

N = 10000
D = 128
DH = 64
E = 320000
NW = 32
NPT = 320
NPAD = NW * NPT
NPS = 16 * NPT
CHUNK = 1600
NGRP = CHUNK // 16
NCHUNK = E // CHUNK
K = 128
PCAP = 8192
NEG = float("-inf")


def _agg_body(nf2, src_h, dst_h, f_h, sum_o, max_o, dir_o,
              dchunk, schunk, fchunk, psrc, pdst, pf,
              idxbuf, dstlbuf, rows, scaled, maxacc, sum_sp, dir_sp,
              dsem, gsem, ssem):
    c = lax.axis_index("c")
    s = lax.axis_index("s")
    tile_lo = (c * 16 + s) * NPT
    sc_base = s * NPT
    dummy_dst = tile_lo + NPT
    lane = lax.iota(jnp.int32, 16)

    def pass_body(h, carry):
        def init_max(i, cy):
            for j in range(DH // 16):
                maxacc[i, pl.ds(j * 16, 16)] = jnp.full((16,), NEG, jnp.float32)
            return cy
        lax.fori_loop(0, NPT + 8, init_max, 0)

        def zero_scaled(i, cy):
            for j in range(DH // 16):
                scaled[0, i, pl.ds(j * 16, 16)] = jnp.zeros((16,), jnp.float32)
            return cy
        lax.fori_loop(0, K, zero_scaled, 0)

        for acc in (sum_sp, dir_sp):
            pltpu.sync_copy(scaled.at[0], acc.at[pl.ds(sc_base, K)])
            pltpu.sync_copy(scaled.at[0], acc.at[pl.ds(sc_base + K, K)])
            pltpu.sync_copy(scaled.at[0, pl.ds(0, NPT - 2 * K)],
                            acc.at[pl.ds(sc_base + 2 * K, NPT - 2 * K)])

        def fill(b):
            par = lax.rem(b, 2)
            for g in range(8):
                sv = psrc[pl.ds(b * K + g * 16, 16)]
                dv = pdst[pl.ds(b * K + g * 16, 16)]
                idxbuf[par, pl.ds(g * 16, 16)] = sv * 2 + h
                dstlbuf[par, pl.ds(g * 16, 16)] = dv - c * NPS

        def issue_gather(b):
            par = lax.rem(b, 2)
            pltpu.async_copy(nf2.at[idxbuf.at[par]], rows.at[par], gsem)

        def wait_gather(b):
            par = lax.rem(b, 2)
            pltpu.make_async_copy(nf2.at[idxbuf.at[par]], rows.at[par],
                                  gsem).wait()

        def wait_scatters(b):
            par = lax.rem(b, 2)
            pltpu.make_async_copy(rows.at[par],
                                  sum_sp.at[dstlbuf.at[par]], ssem).wait()
            pltpu.make_async_copy(scaled.at[par],
                                  dir_sp.at[dstlbuf.at[par]], ssem).wait()

        def compute_batch(b):
            par = lax.rem(b, 2)

            def grp(g, cy):
                dv = pdst[pl.ds(b * K + g * 16, 16)]
                fv16 = pf[pl.ds(b * K + g * 16, 16)]
                for l in range(16):
                    dstl = jnp.max(jnp.where(lane == l, dv, 0)) - tile_lo
                    fsc = jnp.max(jnp.where(lane == l, fv16, NEG))
                    e = g * 16 + l
                    for j in range(DH // 16):
                        r = rows[par, e, pl.ds(j * 16, 16)]
                        scaled[par, e, pl.ds(j * 16, 16)] = r * fsc
                        a = maxacc[dstl, pl.ds(j * 16, 16)]
                        maxacc[dstl, pl.ds(j * 16, 16)] = jnp.maximum(a, r)
                return cy
            lax.fori_loop(0, 8, grp, 0)

        def drain_pipe(p):
            nb = p // K

            @pl.when(nb > 0)
            def _():
                fill(0)
                issue_gather(0)

            def bat(b, cy):
                par = lax.rem(b, 2)

                @pl.when(b >= 1)
                def _():
                    wait_scatters(b - 1)

                @pl.when(b + 1 < nb)
                def _():
                    fill(b + 1)
                    issue_gather(b + 1)
                wait_gather(b)
                pltpu.async_copy(rows.at[par], sum_sp.at[dstlbuf.at[par]],
                                 ssem, add=True)
                compute_batch(b)
                pltpu.async_copy(scaled.at[par], dir_sp.at[dstlbuf.at[par]],
                                 ssem, add=True)
                return cy
            lax.fori_loop(0, nb, bat, 0)

            @pl.when(nb > 0)
            def _():
                wait_scatters(nb - 1)
                def mv(g, cy):
                    sv = psrc[pl.ds(nb * K + g * 16, 16)]
                    dv = pdst[pl.ds(nb * K + g * 16, 16)]
                    fv = pf[pl.ds(nb * K + g * 16, 16)]
                    psrc[pl.ds(g * 16, 16)] = sv
                    pdst[pl.ds(g * 16, 16)] = dv
                    pf[pl.ds(g * 16, 16)] = fv
                    return cy
                lax.fori_loop(0, 8, mv, 0)
            return p - nb * K

        def scan_chunk(par, qv):
            def scan_grp(g, q):
                d16 = dchunk[par, pl.ds(g * 16, 16)]
                msk = (d16 >= tile_lo) & (d16 < tile_lo + NPT)
                s16 = schunk[par, pl.ds(g * 16, 16)]
                f16 = fchunk[par, pl.ds(g * 16, 16)]
                mi = jnp.where(msk, 1, 0).astype(jnp.int32)
                cs = plsc.cumsum(mi)
                pos = cs - mi + q
                plsc.store_scatter(pdst, [pos], d16, mask=msk)
                plsc.store_scatter(psrc, [pos], s16, mask=msk)
                plsc.store_scatter(pf, [pos], f16, mask=msk)
                return q + plsc.all_reduce_population_count(msk)
            return lax.fori_loop(0, NGRP, scan_grp, qv)

        def issue_chunk(i, par):
            pltpu.async_copy(dst_h.at[pl.ds(i * CHUNK, CHUNK)],
                             dchunk.at[par], dsem)
            pltpu.async_copy(src_h.at[pl.ds(i * CHUNK, CHUNK)],
                             schunk.at[par], dsem)
            pltpu.async_copy(f_h.at[pl.ds(i * CHUNK, CHUNK)],
                             fchunk.at[par], dsem)

        def wait_chunk(i, par):
            pltpu.make_async_copy(dst_h.at[pl.ds(i * CHUNK, CHUNK)],
                                  dchunk.at[par], dsem).wait()
            pltpu.make_async_copy(src_h.at[pl.ds(i * CHUNK, CHUNK)],
                                  schunk.at[par], dsem).wait()
            pltpu.make_async_copy(f_h.at[pl.ds(i * CHUNK, CHUNK)],
                                  fchunk.at[par], dsem).wait()

        def chunk_body(i, qv):
            par = lax.rem(i, 2)
            wait_chunk(i, par)

            @pl.when(i + 1 < NCHUNK)
            def _():
                issue_chunk(i + 1, 1 - par)
            qv = scan_chunk(par, qv)
            ps = jnp.max(qv)
            ps = lax.cond(ps >= PCAP - CHUNK, drain_pipe, lambda q: q, ps)
            return jnp.full((16,), 1, jnp.int32) * ps

        issue_chunk(0, 0)
        qv = lax.fori_loop(0, NCHUNK, chunk_body, jnp.zeros((16,), jnp.int32))
        p = jnp.max(qv)

        r = drain_pipe(p)

        @pl.when(r > 0)
        def _():
            dummy_d = jnp.full((16,), 1, jnp.int32) * dummy_dst
            zi = jnp.zeros((16,), jnp.int32)
            zf = jnp.zeros((16,), jnp.float32)
            pdst[pl.ds(r, 16)] = dummy_d
            psrc[pl.ds(r, 16)] = zi
            pf[pl.ds(r, 16)] = zf
            for t in range(1, 8):
                @pl.when(t * 16 >= r)
                def _():
                    pdst[pl.ds(t * 16, 16)] = dummy_d
                    psrc[pl.ds(t * 16, 16)] = zi
                    pf[pl.ds(t * 16, 16)] = zf
            fill(0)
            issue_gather(0)
            wait_gather(0)

            def zrow(e, cy):
                for j in range(DH // 16):
                    rows[0, e, pl.ds(j * 16, 16)] = jnp.zeros((16,), jnp.float32)
                return cy
            lax.fori_loop(r, K, zrow, 0)
            pltpu.sync_copy(rows.at[0], sum_sp.at[dstlbuf.at[0]], add=True)
            compute_batch(0)
            pltpu.sync_copy(scaled.at[0], dir_sp.at[dstlbuf.at[0]], add=True)

        pltpu.sync_copy(maxacc.at[pl.ds(0, NPT)],
                        max_o.at[h, pl.ds(tile_lo, NPT)])
        pltpu.sync_copy(sum_sp.at[pl.ds(sc_base, NPT)],
                        sum_o.at[h, pl.ds(tile_lo, NPT)])
        pltpu.sync_copy(dir_sp.at[pl.ds(sc_base, NPT)],
                        dir_o.at[h, pl.ds(tile_lo, NPT)])
        return carry

    lax.fori_loop(0, 2, pass_body, 0)


_agg = functools.partial(
    pl.kernel,
    mesh=plsc.VectorSubcoreMesh(core_axis_name="c", subcore_axis_name="s"),
    compiler_params=pltpu.CompilerParams(needs_layout_passes=False,
                                         use_tc_tiling_on_sc=False),
    out_type=[
        jax.ShapeDtypeStruct((2, NPAD, DH), jnp.float32),
        jax.ShapeDtypeStruct((2, NPAD, DH), jnp.float32),
        jax.ShapeDtypeStruct((2, NPAD, DH), jnp.float32),
    ],
    scratch_types=[
        pltpu.VMEM((2, CHUNK), jnp.int32),
        pltpu.VMEM((2, CHUNK), jnp.int32),
        pltpu.VMEM((2, CHUNK), jnp.float32),
        pltpu.VMEM((PCAP + 128,), jnp.int32),
        pltpu.VMEM((PCAP + 128,), jnp.int32),
        pltpu.VMEM((PCAP + 128,), jnp.float32),
        pltpu.VMEM((2, K), jnp.int32),
        pltpu.VMEM((2, K), jnp.int32),
        pltpu.VMEM((2, K, DH), jnp.float32),
        pltpu.VMEM((2, K, DH), jnp.float32),
        pltpu.VMEM((NPT + 8, DH), jnp.float32),
        pltpu.VMEM_SHARED((NPS + 8, DH), jnp.float32),
        pltpu.VMEM_SHARED((NPS + 8, DH), jnp.float32),
        pltpu.SemaphoreType.DMA,
        pltpu.SemaphoreType.DMA,
        pltpu.SemaphoreType.DMA,
    ],
)(_agg_body)


N_BLOCK = 1000


def _post_kernel(nf_ref, s_ref, m_ref, dirsum_ref, deg_ref, fdig_ref,
                 norm_ref, w_ref, b_ref, out_ref):
    nf = nf_ref[...]
    s = s_ref[...]
    m = m_ref[...]
    dirsum = dirsum_ref[...]
    deg = deg_ref[...]
    fdig = fdig_ref[...]
    norm = norm_ref[...]
    w = w_ref[...]
    b = b_ref[...]

    mean = s / jnp.maximum(deg, 1.0)
    maxv = jnp.where(jnp.isfinite(m), m, 0.0)
    dirv = dirsum - fdig * nf
    h = jnp.concatenate([nf, mean, maxv, dirv], axis=1)
    out = jnp.dot(h, w, preferred_element_type=jnp.float32) + b[0]
    out_ref[...] = nf + out * norm


def _post_transform(node_fts, s, m, dirsum, deg, fdig, norm_n, W_post, b_post):
    n, d = node_fts.shape
    grid = (n // N_BLOCK,)
    blk = lambda i: (i, 0)
    return pl.pallas_call(
        _post_kernel,
        grid=grid,
        in_specs=[
            pl.BlockSpec((N_BLOCK, d), blk),
            pl.BlockSpec((N_BLOCK, d), blk),
            pl.BlockSpec((N_BLOCK, d), blk),
            pl.BlockSpec((N_BLOCK, d), blk),
            pl.BlockSpec((N_BLOCK, 1), blk),
            pl.BlockSpec((N_BLOCK, 1), blk),
            pl.BlockSpec((N_BLOCK, 1), blk),
            pl.BlockSpec((4 * d, d), lambda i: (0, 0)),
            pl.BlockSpec((1, d), lambda i: (0, 0)),
        ],
        out_specs=pl.BlockSpec((N_BLOCK, d), blk),
        out_shape=jax.ShapeDtypeStruct((n, d), jnp.float32),
    )(node_fts, s, m, dirsum, deg, fdig, norm_n, W_post, b_post)


def kernel(node_fts, edge_fts, edge_index, F_norm_edge, F_dig, node_deg_vec,
           node_deg_mat, lap_mat, k_eig_val, k_eig_vec, num_nodes, norm_n,
           batch_idx, W_post, b_post):
    src = edge_index[0]
    dst = edge_index[1]
    f = F_norm_edge[:, 0]
    nf2 = node_fts.reshape(2 * N, DH)
    s3, m3, dir3 = _agg(nf2, src, dst, f)
    s = jnp.concatenate([s3[0, :N], s3[1, :N]], axis=1)
    m = jnp.concatenate([m3[0, :N], m3[1, :N]], axis=1)
    dirsum = jnp.concatenate([dir3[0, :N], dir3[1, :N]], axis=1)
    return _post_transform(node_fts, s, m, dirsum, node_deg_vec, F_dig,
                           norm_n, W_post, b_post[None, :])


# speedup vs baseline: 2.3153x; 1.0091x over previous
import functools

import jax
import jax.numpy as jnp
from jax import lax
from jax.experimental import pallas as pl
from jax.experimental.pallas import tpu as pltpu
from jax.experimental.pallas import tpu_sc as plsc

N = 10000
D = 128
DH = 64
E = 320000
NW = 32
NPT = 320
NPAD = NW * NPT
NPS = 16 * NPT
CHUNK = 1600
NGRP = CHUNK // 16
NCHUNK = E // CHUNK
K = 128
PCAP = 8192
NEG = float("-inf")


def _agg_body(nf2, src_h, dst_h, f_h, sum_o, max_o, dir_o,
              dchunk, schunk, fchunk, psrc, pdst, pf,
              idxbuf, dstlbuf, rows, scaled, maxacc, sum_sp, dir_sp,
              dsem, gsem, ssem):
    c = lax.axis_index("c")
    s = lax.axis_index("s")
    tile_lo = (c * 16 + s) * NPT
    sc_base = s * NPT
    dummy_dst = tile_lo + NPT
    lane = lax.iota(jnp.int32, 16)

    def pass_body(h, carry):
        def init_max(i, cy):
            for j in range(DH // 16):
                maxacc[i, pl.ds(j * 16, 16)] = jnp.full((16,), NEG, jnp.float32)
            return cy
        lax.fori_loop(0, NPT + 8, init_max, 0)

        def zero_scaled(i, cy):
            for j in range(DH // 16):
                scaled[0, i, pl.ds(j * 16, 16)] = jnp.zeros((16,), jnp.float32)
            return cy
        lax.fori_loop(0, K, zero_scaled, 0)

        for acc in (sum_sp, dir_sp):
            pltpu.sync_copy(scaled.at[0], acc.at[pl.ds(sc_base, K)])
            pltpu.sync_copy(scaled.at[0], acc.at[pl.ds(sc_base + K, K)])
            pltpu.sync_copy(scaled.at[0, pl.ds(0, NPT - 2 * K)],
                            acc.at[pl.ds(sc_base + 2 * K, NPT - 2 * K)])

        def fill(b):
            par = lax.rem(b, 2)
            for g in range(8):
                sv = psrc[pl.ds(b * K + g * 16, 16)]
                dv = pdst[pl.ds(b * K + g * 16, 16)]
                idxbuf[par, pl.ds(g * 16, 16)] = sv * 2 + h
                dstlbuf[par, pl.ds(g * 16, 16)] = dv - c * NPS

        def issue_gather(b):
            par = lax.rem(b, 2)
            pltpu.async_copy(nf2.at[idxbuf.at[par]], rows.at[par], gsem)

        def wait_gather(b):
            par = lax.rem(b, 2)
            pltpu.make_async_copy(nf2.at[idxbuf.at[par]], rows.at[par],
                                  gsem).wait()

        def wait_scatters(b):
            par = lax.rem(b, 2)
            pltpu.make_async_copy(rows.at[par],
                                  sum_sp.at[dstlbuf.at[par]], ssem).wait()
            pltpu.make_async_copy(scaled.at[par],
                                  dir_sp.at[dstlbuf.at[par]], ssem).wait()

        def compute_batch(b):
            par = lax.rem(b, 2)

            def grp(g, cy):
                dv = pdst[pl.ds(b * K + g * 16, 16)]
                fv16 = pf[pl.ds(b * K + g * 16, 16)]
                for l in range(16):
                    dstl = jnp.max(jnp.where(lane == l, dv, 0)) - tile_lo
                    fsc = jnp.max(jnp.where(lane == l, fv16, NEG))
                    e = g * 16 + l
                    for j in range(DH // 16):
                        r = rows[par, e, pl.ds(j * 16, 16)]
                        scaled[par, e, pl.ds(j * 16, 16)] = r * fsc
                        a = maxacc[dstl, pl.ds(j * 16, 16)]
                        maxacc[dstl, pl.ds(j * 16, 16)] = jnp.maximum(a, r)
                return cy
            lax.fori_loop(0, 8, grp, 0)

        def drain_pipe(p):
            nb = p // K

            @pl.when(nb > 0)
            def _():
                fill(0)
                issue_gather(0)

            def bat(b, cy):
                par = lax.rem(b, 2)

                @pl.when(b >= 1)
                def _():
                    wait_scatters(b - 1)

                @pl.when(b + 1 < nb)
                def _():
                    fill(b + 1)
                    issue_gather(b + 1)
                wait_gather(b)
                pltpu.async_copy(rows.at[par], sum_sp.at[dstlbuf.at[par]],
                                 ssem, add=True)
                compute_batch(b)
                pltpu.async_copy(scaled.at[par], dir_sp.at[dstlbuf.at[par]],
                                 ssem, add=True)
                return cy
            lax.fori_loop(0, nb, bat, 0)

            @pl.when(nb > 0)
            def _():
                wait_scatters(nb - 1)
                def mv(g, cy):
                    sv = psrc[pl.ds(nb * K + g * 16, 16)]
                    dv = pdst[pl.ds(nb * K + g * 16, 16)]
                    fv = pf[pl.ds(nb * K + g * 16, 16)]
                    psrc[pl.ds(g * 16, 16)] = sv
                    pdst[pl.ds(g * 16, 16)] = dv
                    pf[pl.ds(g * 16, 16)] = fv
                    return cy
                lax.fori_loop(0, 8, mv, 0)
            return p - nb * K

        def scan_chunk(par, qv):
            def scan_grp4(g4, q):
                for u in range(4):
                    off = g4 * 64 + u * 16
                    d16 = dchunk[par, pl.ds(off, 16)]
                    msk = (d16 >= tile_lo) & (d16 < tile_lo + NPT)
                    s16 = schunk[par, pl.ds(off, 16)]
                    f16 = fchunk[par, pl.ds(off, 16)]
                    mi = jnp.where(msk, 1, 0).astype(jnp.int32)
                    cs = plsc.cumsum(mi)
                    pos = cs - mi + q
                    plsc.store_scatter(pdst, [pos], d16, mask=msk)
                    plsc.store_scatter(psrc, [pos], s16, mask=msk)
                    plsc.store_scatter(pf, [pos], f16, mask=msk)
                    q = q + plsc.all_reduce_population_count(msk)
                return q
            return lax.fori_loop(0, NGRP // 4, scan_grp4, qv)

        def issue_chunk(i, par):
            pltpu.async_copy(dst_h.at[pl.ds(i * CHUNK, CHUNK)],
                             dchunk.at[par], dsem)
            pltpu.async_copy(src_h.at[pl.ds(i * CHUNK, CHUNK)],
                             schunk.at[par], dsem)
            pltpu.async_copy(f_h.at[pl.ds(i * CHUNK, CHUNK)],
                             fchunk.at[par], dsem)

        def wait_chunk(i, par):
            pltpu.make_async_copy(dst_h.at[pl.ds(i * CHUNK, CHUNK)],
                                  dchunk.at[par], dsem).wait()
            pltpu.make_async_copy(src_h.at[pl.ds(i * CHUNK, CHUNK)],
                                  schunk.at[par], dsem).wait()
            pltpu.make_async_copy(f_h.at[pl.ds(i * CHUNK, CHUNK)],
                                  fchunk.at[par], dsem).wait()

        def chunk_body(i, qv):
            par = lax.rem(i, 2)
            wait_chunk(i, par)

            @pl.when(i + 1 < NCHUNK)
            def _():
                issue_chunk(i + 1, 1 - par)
            qv = scan_chunk(par, qv)
            ps = jnp.max(qv)
            ps = lax.cond(ps >= PCAP - CHUNK, drain_pipe, lambda q: q, ps)
            return jnp.full((16,), 1, jnp.int32) * ps

        issue_chunk(0, 0)
        qv = lax.fori_loop(0, NCHUNK, chunk_body, jnp.zeros((16,), jnp.int32))
        p = jnp.max(qv)

        r = drain_pipe(p)

        @pl.when(r > 0)
        def _():
            dummy_d = jnp.full((16,), 1, jnp.int32) * dummy_dst
            zi = jnp.zeros((16,), jnp.int32)
            zf = jnp.zeros((16,), jnp.float32)
            pdst[pl.ds(r, 16)] = dummy_d
            psrc[pl.ds(r, 16)] = zi
            pf[pl.ds(r, 16)] = zf
            for t in range(1, 8):
                @pl.when(t * 16 >= r)
                def _():
                    pdst[pl.ds(t * 16, 16)] = dummy_d
                    psrc[pl.ds(t * 16, 16)] = zi
                    pf[pl.ds(t * 16, 16)] = zf
            fill(0)
            issue_gather(0)
            wait_gather(0)

            def zrow(e, cy):
                for j in range(DH // 16):
                    rows[0, e, pl.ds(j * 16, 16)] = jnp.zeros((16,), jnp.float32)
                return cy
            lax.fori_loop(r, K, zrow, 0)
            pltpu.sync_copy(rows.at[0], sum_sp.at[dstlbuf.at[0]], add=True)
            compute_batch(0)
            pltpu.sync_copy(scaled.at[0], dir_sp.at[dstlbuf.at[0]], add=True)

        pltpu.sync_copy(maxacc.at[pl.ds(0, NPT)],
                        max_o.at[h, pl.ds(tile_lo, NPT)])
        pltpu.sync_copy(sum_sp.at[pl.ds(sc_base, NPT)],
                        sum_o.at[h, pl.ds(tile_lo, NPT)])
        pltpu.sync_copy(dir_sp.at[pl.ds(sc_base, NPT)],
                        dir_o.at[h, pl.ds(tile_lo, NPT)])
        return carry

    lax.fori_loop(0, 2, pass_body, 0)


_agg = functools.partial(
    pl.kernel,
    mesh=plsc.VectorSubcoreMesh(core_axis_name="c", subcore_axis_name="s"),
    compiler_params=pltpu.CompilerParams(needs_layout_passes=False,
                                         use_tc_tiling_on_sc=False),
    out_type=[
        jax.ShapeDtypeStruct((2, NPAD, DH), jnp.float32),
        jax.ShapeDtypeStruct((2, NPAD, DH), jnp.float32),
        jax.ShapeDtypeStruct((2, NPAD, DH), jnp.float32),
    ],
    scratch_types=[
        pltpu.VMEM((2, CHUNK), jnp.int32),
        pltpu.VMEM((2, CHUNK), jnp.int32),
        pltpu.VMEM((2, CHUNK), jnp.float32),
        pltpu.VMEM((PCAP + 128,), jnp.int32),
        pltpu.VMEM((PCAP + 128,), jnp.int32),
        pltpu.VMEM((PCAP + 128,), jnp.float32),
        pltpu.VMEM((2, K), jnp.int32),
        pltpu.VMEM((2, K), jnp.int32),
        pltpu.VMEM((2, K, DH), jnp.float32),
        pltpu.VMEM((2, K, DH), jnp.float32),
        pltpu.VMEM((NPT + 8, DH), jnp.float32),
        pltpu.VMEM_SHARED((NPS + 8, DH), jnp.float32),
        pltpu.VMEM_SHARED((NPS + 8, DH), jnp.float32),
        pltpu.SemaphoreType.DMA,
        pltpu.SemaphoreType.DMA,
        pltpu.SemaphoreType.DMA,
    ],
)(_agg_body)


N_BLOCK = 1000


def _post_kernel(nf_ref, s_ref, m_ref, dirsum_ref, deg_ref, fdig_ref,
                 norm_ref, w_ref, b_ref, out_ref):
    nf = nf_ref[...]
    s = s_ref[...]
    m = m_ref[...]
    dirsum = dirsum_ref[...]
    deg = deg_ref[...]
    fdig = fdig_ref[...]
    norm = norm_ref[...]
    w = w_ref[...]
    b = b_ref[...]

    mean = s / jnp.maximum(deg, 1.0)
    maxv = jnp.where(jnp.isfinite(m), m, 0.0)
    dirv = dirsum - fdig * nf
    h = jnp.concatenate([nf, mean, maxv, dirv], axis=1)
    out = jnp.dot(h, w, preferred_element_type=jnp.float32) + b[0]
    out_ref[...] = nf + out * norm


def _post_transform(node_fts, s, m, dirsum, deg, fdig, norm_n, W_post, b_post):
    n, d = node_fts.shape
    grid = (n // N_BLOCK,)
    blk = lambda i: (i, 0)
    return pl.pallas_call(
        _post_kernel,
        grid=grid,
        in_specs=[
            pl.BlockSpec((N_BLOCK, d), blk),
            pl.BlockSpec((N_BLOCK, d), blk),
            pl.BlockSpec((N_BLOCK, d), blk),
            pl.BlockSpec((N_BLOCK, d), blk),
            pl.BlockSpec((N_BLOCK, 1), blk),
            pl.BlockSpec((N_BLOCK, 1), blk),
            pl.BlockSpec((N_BLOCK, 1), blk),
            pl.BlockSpec((4 * d, d), lambda i: (0, 0)),
            pl.BlockSpec((1, d), lambda i: (0, 0)),
        ],
        out_specs=pl.BlockSpec((N_BLOCK, d), blk),
        out_shape=jax.ShapeDtypeStruct((n, d), jnp.float32),
    )(node_fts, s, m, dirsum, deg, fdig, norm_n, W_post, b_post)


def kernel(node_fts, edge_fts, edge_index, F_norm_edge, F_dig, node_deg_vec,
           node_deg_mat, lap_mat, k_eig_val, k_eig_vec, num_nodes, norm_n,
           batch_idx, W_post, b_post):
    src = edge_index[0]
    dst = edge_index[1]
    f = F_norm_edge[:, 0]
    nf2 = node_fts.reshape(2 * N, DH)
    s3, m3, dir3 = _agg(nf2, src, dst, f)
    s = jnp.concatenate([s3[0, :N], s3[1, :N]], axis=1)
    m = jnp.concatenate([m3[0, :N], m3[1, :N]], axis=1)
    dirsum = jnp.concatenate([dir3[0, :N], dir3[1, :N]], axis=1)
    return _post_transform(node_fts, s, m, dirsum, node_deg_vec, F_dig,
                           norm_n, W_post, b_post[None, :])


# speedup vs baseline: 2.3797x; 1.0278x over previous
import functools

import jax
import jax.numpy as jnp
from jax import lax
from jax.experimental import pallas as pl
from jax.experimental.pallas import tpu as pltpu
from jax.experimental.pallas import tpu_sc as plsc

N = 10000
D = 128
DH = 64
E = 320000
NW = 32
NPT = 320
NPAD = NW * NPT
NPS = 16 * NPT
CHUNK = 1600
NGRP = CHUNK // 16
NCHUNK = E // CHUNK
K = 128
PCAP = 4096
NEG = float("-inf")


def _agg_body(nf2, src_h, dst_h, f_h, sum_o, max_o, dir_o,
              dchunk, schunk, fchunk, psrc, pdst, pf,
              idxbuf, dstlbuf, rows, scaled, maxacc, sum_sp, dir_sp,
              dsem, gsem, ssem):
    c = lax.axis_index("c")
    s = lax.axis_index("s")
    tile_lo = (c * 16 + s) * NPT
    sc_base = s * NPT
    dummy_dst = tile_lo + NPT
    lane = lax.iota(jnp.int32, 16)

    def pass_body(h, carry):
        def init_max(i, cy):
            for j in range(DH // 16):
                maxacc[i, pl.ds(j * 16, 16)] = jnp.full((16,), NEG, jnp.float32)
            return cy
        lax.fori_loop(0, NPT + 8, init_max, 0)

        def zero_scaled(i, cy):
            for j in range(DH // 16):
                scaled[0, i, pl.ds(j * 16, 16)] = jnp.zeros((16,), jnp.float32)
            return cy
        lax.fori_loop(0, K, zero_scaled, 0)

        for acc in (sum_sp, dir_sp):
            pltpu.sync_copy(scaled.at[0], acc.at[pl.ds(sc_base, K)])
            pltpu.sync_copy(scaled.at[0], acc.at[pl.ds(sc_base + K, K)])
            pltpu.sync_copy(scaled.at[0, pl.ds(0, NPT - 2 * K)],
                            acc.at[pl.ds(sc_base + 2 * K, NPT - 2 * K)])

        def fill(b):
            par = lax.rem(b, 3)
            for g in range(8):
                sv = psrc[pl.ds(b * K + g * 16, 16)]
                dv = pdst[pl.ds(b * K + g * 16, 16)]
                idxbuf[par, pl.ds(g * 16, 16)] = sv * 2 + h
                dstlbuf[par, pl.ds(g * 16, 16)] = dv - c * NPS

        def issue_gather(b):
            par = lax.rem(b, 3)
            pltpu.async_copy(nf2.at[idxbuf.at[par]], rows.at[par], gsem)

        def wait_gather(b):
            par = lax.rem(b, 3)
            pltpu.make_async_copy(nf2.at[idxbuf.at[par]], rows.at[par],
                                  gsem).wait()

        def wait_scatters(b):
            par = lax.rem(b, 3)
            par2 = lax.rem(b, 2)
            pltpu.make_async_copy(rows.at[par],
                                  sum_sp.at[dstlbuf.at[par]], ssem).wait()
            pltpu.make_async_copy(scaled.at[par2],
                                  dir_sp.at[dstlbuf.at[par]], ssem).wait()

        def compute_batch(b):
            par = lax.rem(b, 3)
            par2 = lax.rem(b, 2)

            def grp(g, cy):
                dv = pdst[pl.ds(b * K + g * 16, 16)]
                fv16 = pf[pl.ds(b * K + g * 16, 16)]
                for l in range(16):
                    dstl = jnp.max(jnp.where(lane == l, dv, 0)) - tile_lo
                    fsc = jnp.max(jnp.where(lane == l, fv16, NEG))
                    e = g * 16 + l
                    for j in range(DH // 16):
                        r = rows[par, e, pl.ds(j * 16, 16)]
                        scaled[par2, e, pl.ds(j * 16, 16)] = r * fsc
                        a = maxacc[dstl, pl.ds(j * 16, 16)]
                        maxacc[dstl, pl.ds(j * 16, 16)] = jnp.maximum(a, r)
                return cy
            lax.fori_loop(0, 8, grp, 0)

        def drain_pipe(p):
            nb = p // K

            @pl.when(nb > 0)
            def _():
                fill(0)
                issue_gather(0)

            @pl.when(nb > 1)
            def _():
                fill(1)
                issue_gather(1)

            def bat(b, cy):
                par = lax.rem(b, 3)
                par2 = lax.rem(b, 2)
                wait_gather(b)
                pltpu.async_copy(rows.at[par], sum_sp.at[dstlbuf.at[par]],
                                 ssem, add=True)
                compute_batch(b)
                pltpu.async_copy(scaled.at[par2], dir_sp.at[dstlbuf.at[par]],
                                 ssem, add=True)

                @pl.when(b >= 1)
                def _():
                    wait_scatters(b - 1)

                @pl.when(b + 2 < nb)
                def _():
                    fill(b + 2)
                    issue_gather(b + 2)
                return cy
            lax.fori_loop(0, nb, bat, 0)

            @pl.when(nb > 0)
            def _():
                wait_scatters(nb - 1)
                def mv(g, cy):
                    sv = psrc[pl.ds(nb * K + g * 16, 16)]
                    dv = pdst[pl.ds(nb * K + g * 16, 16)]
                    fv = pf[pl.ds(nb * K + g * 16, 16)]
                    psrc[pl.ds(g * 16, 16)] = sv
                    pdst[pl.ds(g * 16, 16)] = dv
                    pf[pl.ds(g * 16, 16)] = fv
                    return cy
                lax.fori_loop(0, 8, mv, 0)
            return p - nb * K

        def scan_chunk(par, qv):
            def scan_grp4(g4, q):
                for u in range(4):
                    off = g4 * 64 + u * 16
                    d16 = dchunk[par, pl.ds(off, 16)]
                    msk = (d16 >= tile_lo) & (d16 < tile_lo + NPT)
                    s16 = schunk[par, pl.ds(off, 16)]
                    f16 = fchunk[par, pl.ds(off, 16)]
                    mi = jnp.where(msk, 1, 0).astype(jnp.int32)
                    cs = plsc.cumsum(mi)
                    pos = cs - mi + q
                    plsc.store_scatter(pdst, [pos], d16, mask=msk)
                    plsc.store_scatter(psrc, [pos], s16, mask=msk)
                    plsc.store_scatter(pf, [pos], f16, mask=msk)
                    q = q + plsc.all_reduce_population_count(msk)
                return q
            return lax.fori_loop(0, NGRP // 4, scan_grp4, qv)

        def issue_chunk(i, par):
            pltpu.async_copy(dst_h.at[pl.ds(i * CHUNK, CHUNK)],
                             dchunk.at[par], dsem)
            pltpu.async_copy(src_h.at[pl.ds(i * CHUNK, CHUNK)],
                             schunk.at[par], dsem)
            pltpu.async_copy(f_h.at[pl.ds(i * CHUNK, CHUNK)],
                             fchunk.at[par], dsem)

        def wait_chunk(i, par):
            pltpu.make_async_copy(dst_h.at[pl.ds(i * CHUNK, CHUNK)],
                                  dchunk.at[par], dsem).wait()
            pltpu.make_async_copy(src_h.at[pl.ds(i * CHUNK, CHUNK)],
                                  schunk.at[par], dsem).wait()
            pltpu.make_async_copy(f_h.at[pl.ds(i * CHUNK, CHUNK)],
                                  fchunk.at[par], dsem).wait()

        def chunk_body(i, qv):
            par = lax.rem(i, 2)
            wait_chunk(i, par)

            @pl.when(i + 1 < NCHUNK)
            def _():
                issue_chunk(i + 1, 1 - par)
            qv = scan_chunk(par, qv)
            ps = jnp.max(qv)
            ps = lax.cond(ps >= PCAP - CHUNK, drain_pipe, lambda q: q, ps)
            return jnp.full((16,), 1, jnp.int32) * ps

        issue_chunk(0, 0)
        qv = lax.fori_loop(0, NCHUNK, chunk_body, jnp.zeros((16,), jnp.int32))
        p = jnp.max(qv)

        r = drain_pipe(p)

        @pl.when(r > 0)
        def _():
            dummy_d = jnp.full((16,), 1, jnp.int32) * dummy_dst
            zi = jnp.zeros((16,), jnp.int32)
            zf = jnp.zeros((16,), jnp.float32)
            pdst[pl.ds(r, 16)] = dummy_d
            psrc[pl.ds(r, 16)] = zi
            pf[pl.ds(r, 16)] = zf
            for t in range(1, 8):
                @pl.when(t * 16 >= r)
                def _():
                    pdst[pl.ds(t * 16, 16)] = dummy_d
                    psrc[pl.ds(t * 16, 16)] = zi
                    pf[pl.ds(t * 16, 16)] = zf
            fill(0)
            issue_gather(0)
            wait_gather(0)

            def zrow(e, cy):
                for j in range(DH // 16):
                    rows[0, e, pl.ds(j * 16, 16)] = jnp.zeros((16,), jnp.float32)
                return cy
            lax.fori_loop(r, K, zrow, 0)
            pltpu.sync_copy(rows.at[0], sum_sp.at[dstlbuf.at[0]], add=True)
            compute_batch(0)
            pltpu.sync_copy(scaled.at[0], dir_sp.at[dstlbuf.at[0]], add=True)

        pltpu.sync_copy(maxacc.at[pl.ds(0, NPT)],
                        max_o.at[h, pl.ds(tile_lo, NPT)])
        pltpu.sync_copy(sum_sp.at[pl.ds(sc_base, NPT)],
                        sum_o.at[h, pl.ds(tile_lo, NPT)])
        pltpu.sync_copy(dir_sp.at[pl.ds(sc_base, NPT)],
                        dir_o.at[h, pl.ds(tile_lo, NPT)])
        return carry

    lax.fori_loop(0, 2, pass_body, 0)


_agg = functools.partial(
    pl.kernel,
    mesh=plsc.VectorSubcoreMesh(core_axis_name="c", subcore_axis_name="s"),
    compiler_params=pltpu.CompilerParams(needs_layout_passes=False,
                                         use_tc_tiling_on_sc=False),
    out_type=[
        jax.ShapeDtypeStruct((2, NPAD, DH), jnp.float32),
        jax.ShapeDtypeStruct((2, NPAD, DH), jnp.float32),
        jax.ShapeDtypeStruct((2, NPAD, DH), jnp.float32),
    ],
    scratch_types=[
        pltpu.VMEM((2, CHUNK), jnp.int32),
        pltpu.VMEM((2, CHUNK), jnp.int32),
        pltpu.VMEM((2, CHUNK), jnp.float32),
        pltpu.VMEM((PCAP + 128,), jnp.int32),
        pltpu.VMEM((PCAP + 128,), jnp.int32),
        pltpu.VMEM((PCAP + 128,), jnp.float32),
        pltpu.VMEM((3, K), jnp.int32),
        pltpu.VMEM((3, K), jnp.int32),
        pltpu.VMEM((3, K, DH), jnp.float32),
        pltpu.VMEM((2, K, DH), jnp.float32),
        pltpu.VMEM((NPT + 8, DH), jnp.float32),
        pltpu.VMEM_SHARED((NPS + 8, DH), jnp.float32),
        pltpu.VMEM_SHARED((NPS + 8, DH), jnp.float32),
        pltpu.SemaphoreType.DMA,
        pltpu.SemaphoreType.DMA,
        pltpu.SemaphoreType.DMA,
    ],
)(_agg_body)


N_BLOCK = 1000


def _post_kernel(nf_ref, s_ref, m_ref, dirsum_ref, deg_ref, fdig_ref,
                 norm_ref, w_ref, b_ref, out_ref):
    nf = nf_ref[...]
    s = s_ref[...]
    m = m_ref[...]
    dirsum = dirsum_ref[...]
    deg = deg_ref[...]
    fdig = fdig_ref[...]
    norm = norm_ref[...]
    w = w_ref[...]
    b = b_ref[...]

    mean = s / jnp.maximum(deg, 1.0)
    maxv = jnp.where(jnp.isfinite(m), m, 0.0)
    dirv = dirsum - fdig * nf
    h = jnp.concatenate([nf, mean, maxv, dirv], axis=1)
    out = jnp.dot(h, w, preferred_element_type=jnp.float32) + b[0]
    out_ref[...] = nf + out * norm


def _post_transform(node_fts, s, m, dirsum, deg, fdig, norm_n, W_post, b_post):
    n, d = node_fts.shape
    grid = (n // N_BLOCK,)
    blk = lambda i: (i, 0)
    return pl.pallas_call(
        _post_kernel,
        grid=grid,
        in_specs=[
            pl.BlockSpec((N_BLOCK, d), blk),
            pl.BlockSpec((N_BLOCK, d), blk),
            pl.BlockSpec((N_BLOCK, d), blk),
            pl.BlockSpec((N_BLOCK, d), blk),
            pl.BlockSpec((N_BLOCK, 1), blk),
            pl.BlockSpec((N_BLOCK, 1), blk),
            pl.BlockSpec((N_BLOCK, 1), blk),
            pl.BlockSpec((4 * d, d), lambda i: (0, 0)),
            pl.BlockSpec((1, d), lambda i: (0, 0)),
        ],
        out_specs=pl.BlockSpec((N_BLOCK, d), blk),
        out_shape=jax.ShapeDtypeStruct((n, d), jnp.float32),
    )(node_fts, s, m, dirsum, deg, fdig, norm_n, W_post, b_post)


def kernel(node_fts, edge_fts, edge_index, F_norm_edge, F_dig, node_deg_vec,
           node_deg_mat, lap_mat, k_eig_val, k_eig_vec, num_nodes, norm_n,
           batch_idx, W_post, b_post):
    src = edge_index[0]
    dst = edge_index[1]
    f = F_norm_edge[:, 0]
    nf2 = node_fts.reshape(2 * N, DH)
    s3, m3, dir3 = _agg(nf2, src, dst, f)
    s = jnp.concatenate([s3[0, :N], s3[1, :N]], axis=1)
    m = jnp.concatenate([m3[0, :N], m3[1, :N]], axis=1)
    dirsum = jnp.concatenate([dir3[0, :N], dir3[1, :N]], axis=1)
    return _post_transform(node_fts, s, m, dirsum, node_deg_vec, F_dig,
                           norm_n, W_post, b_post[None, :])


# speedup vs baseline: 3.0012x; 1.2612x over previous
import functools

import jax
import jax.numpy as jnp
from jax import lax
from jax.experimental import pallas as pl
from jax.experimental.pallas import tpu as pltpu
from jax.experimental.pallas import tpu_sc as plsc

N = 10000
D = 128
DH = 64
E = 320000
NW = 32
NPT = 320
NPAD = NW * NPT
NPS = 16 * NPT
CHUNK = 1600
NGRP = CHUNK // 16
NCHUNK = E // CHUNK
K = 128
PCAP = 4096
NEG = float("-inf")


def _agg_body(nf2, src_h, dst_h, f_h, sum_o, max_o, dir_o,
              dchunk, schunk, fchunk, psrc, pdst, pf,
              idxbuf, dstlbuf, rows, maxacc, diracc, sum_sp,
              dsem, gsem, ssem):
    c = lax.axis_index("c")
    s = lax.axis_index("s")
    tile_lo = (c * 16 + s) * NPT
    sc_base = s * NPT
    dummy_dst = tile_lo + NPT
    lane = lax.iota(jnp.int32, 16)

    def pass_body(h, carry):
        def init_max(i, cy):
            for j in range(DH // 16):
                maxacc[i, pl.ds(j * 16, 16)] = jnp.full((16,), NEG, jnp.float32)
                diracc[i, pl.ds(j * 16, 16)] = jnp.zeros((16,), jnp.float32)
            return cy
        lax.fori_loop(0, NPT + 8, init_max, 0)

        def zero_rows0(i, cy):
            for j in range(DH // 16):
                rows[0, i, pl.ds(j * 16, 16)] = jnp.zeros((16,), jnp.float32)
            return cy
        lax.fori_loop(0, K, zero_rows0, 0)

        pltpu.sync_copy(rows.at[0], sum_sp.at[pl.ds(sc_base, K)])
        pltpu.sync_copy(rows.at[0], sum_sp.at[pl.ds(sc_base + K, K)])
        pltpu.sync_copy(rows.at[0, pl.ds(0, NPT - 2 * K)],
                        sum_sp.at[pl.ds(sc_base + 2 * K, NPT - 2 * K)])

        def fill(b):
            par = lax.rem(b, 3)
            for g in range(8):
                sv = psrc[pl.ds(b * K + g * 16, 16)]
                dv = pdst[pl.ds(b * K + g * 16, 16)]
                idxbuf[par, pl.ds(g * 16, 16)] = sv * 2 + h
                dstlbuf[par, pl.ds(g * 16, 16)] = dv - c * NPS

        def issue_gather(b):
            par = lax.rem(b, 3)
            pltpu.async_copy(nf2.at[idxbuf.at[par]], rows.at[par], gsem)

        def wait_gather(b):
            par = lax.rem(b, 3)
            pltpu.make_async_copy(nf2.at[idxbuf.at[par]], rows.at[par],
                                  gsem).wait()

        def wait_scatters(b):
            par = lax.rem(b, 3)
            pltpu.make_async_copy(rows.at[par],
                                  sum_sp.at[dstlbuf.at[par]], ssem).wait()

        def compute_batch(b):
            par = lax.rem(b, 3)

            def grp(g, cy):
                dv = pdst[pl.ds(b * K + g * 16, 16)]
                fv16 = pf[pl.ds(b * K + g * 16, 16)]
                for l in range(16):
                    dstl = jnp.max(jnp.where(lane == l, dv, 0)) - tile_lo
                    fsc = jnp.max(jnp.where(lane == l, fv16, NEG))
                    e = g * 16 + l
                    for j in range(DH // 16):
                        r = rows[par, e, pl.ds(j * 16, 16)]
                        plsc.addupdate(diracc.at[dstl, pl.ds(j * 16, 16)],
                                       r * fsc)
                        a = maxacc[dstl, pl.ds(j * 16, 16)]
                        maxacc[dstl, pl.ds(j * 16, 16)] = jnp.maximum(a, r)
                return cy
            lax.fori_loop(0, 8, grp, 0)

        def drain_pipe(p):
            nb = p // K

            @pl.when(nb > 0)
            def _():
                fill(0)
                issue_gather(0)

            @pl.when(nb > 1)
            def _():
                fill(1)
                issue_gather(1)

            def bat(b, cy):
                par = lax.rem(b, 3)
                wait_gather(b)
                pltpu.async_copy(rows.at[par], sum_sp.at[dstlbuf.at[par]],
                                 ssem, add=True)
                compute_batch(b)

                @pl.when(b >= 1)
                def _():
                    wait_scatters(b - 1)

                @pl.when(b + 2 < nb)
                def _():
                    fill(b + 2)
                    issue_gather(b + 2)
                return cy
            lax.fori_loop(0, nb, bat, 0)

            @pl.when(nb > 0)
            def _():
                wait_scatters(nb - 1)
                def mv(g, cy):
                    sv = psrc[pl.ds(nb * K + g * 16, 16)]
                    dv = pdst[pl.ds(nb * K + g * 16, 16)]
                    fv = pf[pl.ds(nb * K + g * 16, 16)]
                    psrc[pl.ds(g * 16, 16)] = sv
                    pdst[pl.ds(g * 16, 16)] = dv
                    pf[pl.ds(g * 16, 16)] = fv
                    return cy
                lax.fori_loop(0, 8, mv, 0)
            return p - nb * K

        def scan_chunk(par, qv):
            def scan_grp4(g4, q):
                for u in range(4):
                    off = g4 * 64 + u * 16
                    d16 = dchunk[par, pl.ds(off, 16)]
                    msk = (d16 >= tile_lo) & (d16 < tile_lo + NPT)
                    s16 = schunk[par, pl.ds(off, 16)]
                    f16 = fchunk[par, pl.ds(off, 16)]
                    mi = jnp.where(msk, 1, 0).astype(jnp.int32)
                    cs = plsc.cumsum(mi)
                    pos = cs - mi + q
                    plsc.store_scatter(pdst, [pos], d16, mask=msk)
                    plsc.store_scatter(psrc, [pos], s16, mask=msk)
                    plsc.store_scatter(pf, [pos], f16, mask=msk)
                    q = q + plsc.all_reduce_population_count(msk)
                return q
            return lax.fori_loop(0, NGRP // 4, scan_grp4, qv)

        def issue_chunk(i, par):
            pltpu.async_copy(dst_h.at[pl.ds(i * CHUNK, CHUNK)],
                             dchunk.at[par], dsem)
            pltpu.async_copy(src_h.at[pl.ds(i * CHUNK, CHUNK)],
                             schunk.at[par], dsem)
            pltpu.async_copy(f_h.at[pl.ds(i * CHUNK, CHUNK)],
                             fchunk.at[par], dsem)

        def wait_chunk(i, par):
            pltpu.make_async_copy(dst_h.at[pl.ds(i * CHUNK, CHUNK)],
                                  dchunk.at[par], dsem).wait()
            pltpu.make_async_copy(src_h.at[pl.ds(i * CHUNK, CHUNK)],
                                  schunk.at[par], dsem).wait()
            pltpu.make_async_copy(f_h.at[pl.ds(i * CHUNK, CHUNK)],
                                  fchunk.at[par], dsem).wait()

        def chunk_body(i, qv):
            par = lax.rem(i, 2)
            wait_chunk(i, par)

            @pl.when(i + 1 < NCHUNK)
            def _():
                issue_chunk(i + 1, 1 - par)
            qv = scan_chunk(par, qv)
            ps = jnp.max(qv)
            ps = lax.cond(ps >= PCAP - CHUNK, drain_pipe, lambda q: q, ps)
            return jnp.full((16,), 1, jnp.int32) * ps

        issue_chunk(0, 0)
        qv = lax.fori_loop(0, NCHUNK, chunk_body, jnp.zeros((16,), jnp.int32))
        p = jnp.max(qv)

        r = drain_pipe(p)

        @pl.when(r > 0)
        def _():
            dummy_d = jnp.full((16,), 1, jnp.int32) * dummy_dst
            zi = jnp.zeros((16,), jnp.int32)
            zf = jnp.zeros((16,), jnp.float32)
            pdst[pl.ds(r, 16)] = dummy_d
            psrc[pl.ds(r, 16)] = zi
            pf[pl.ds(r, 16)] = zf
            for t in range(1, 8):
                @pl.when(t * 16 >= r)
                def _():
                    pdst[pl.ds(t * 16, 16)] = dummy_d
                    psrc[pl.ds(t * 16, 16)] = zi
                    pf[pl.ds(t * 16, 16)] = zf
            fill(0)
            issue_gather(0)
            wait_gather(0)

            def zrow(e, cy):
                for j in range(DH // 16):
                    rows[0, e, pl.ds(j * 16, 16)] = jnp.zeros((16,), jnp.float32)
                return cy
            lax.fori_loop(r, K, zrow, 0)
            pltpu.sync_copy(rows.at[0], sum_sp.at[dstlbuf.at[0]], add=True)
            compute_batch(0)

        pltpu.sync_copy(maxacc.at[pl.ds(0, NPT)],
                        max_o.at[h, pl.ds(tile_lo, NPT)])
        pltpu.sync_copy(sum_sp.at[pl.ds(sc_base, NPT)],
                        sum_o.at[h, pl.ds(tile_lo, NPT)])
        pltpu.sync_copy(diracc.at[pl.ds(0, NPT)],
                        dir_o.at[h, pl.ds(tile_lo, NPT)])
        return carry

    lax.fori_loop(0, 2, pass_body, 0)


_agg = functools.partial(
    pl.kernel,
    mesh=plsc.VectorSubcoreMesh(core_axis_name="c", subcore_axis_name="s"),
    compiler_params=pltpu.CompilerParams(needs_layout_passes=False,
                                         use_tc_tiling_on_sc=False),
    out_type=[
        jax.ShapeDtypeStruct((2, NPAD, DH), jnp.float32),
        jax.ShapeDtypeStruct((2, NPAD, DH), jnp.float32),
        jax.ShapeDtypeStruct((2, NPAD, DH), jnp.float32),
    ],
    scratch_types=[
        pltpu.VMEM((2, CHUNK), jnp.int32),
        pltpu.VMEM((2, CHUNK), jnp.int32),
        pltpu.VMEM((2, CHUNK), jnp.float32),
        pltpu.VMEM((PCAP + 128,), jnp.int32),
        pltpu.VMEM((PCAP + 128,), jnp.int32),
        pltpu.VMEM((PCAP + 128,), jnp.float32),
        pltpu.VMEM((3, K), jnp.int32),
        pltpu.VMEM((3, K), jnp.int32),
        pltpu.VMEM((3, K, DH), jnp.float32),
        pltpu.VMEM((NPT + 8, DH), jnp.float32),
        pltpu.VMEM((NPT + 8, DH), jnp.float32),
        pltpu.VMEM_SHARED((NPS + 8, DH), jnp.float32),
        pltpu.SemaphoreType.DMA,
        pltpu.SemaphoreType.DMA,
        pltpu.SemaphoreType.DMA,
    ],
)(_agg_body)


N_BLOCK = 1000


def _post_kernel(nf_ref, s_ref, m_ref, dirsum_ref, deg_ref, fdig_ref,
                 norm_ref, w_ref, b_ref, out_ref):
    nf = nf_ref[...]
    s = s_ref[...]
    m = m_ref[...]
    dirsum = dirsum_ref[...]
    deg = deg_ref[...]
    fdig = fdig_ref[...]
    norm = norm_ref[...]
    w = w_ref[...]
    b = b_ref[...]

    mean = s / jnp.maximum(deg, 1.0)
    maxv = jnp.where(jnp.isfinite(m), m, 0.0)
    dirv = dirsum - fdig * nf
    h = jnp.concatenate([nf, mean, maxv, dirv], axis=1)
    out = jnp.dot(h, w, preferred_element_type=jnp.float32) + b[0]
    out_ref[...] = nf + out * norm


def _post_transform(node_fts, s, m, dirsum, deg, fdig, norm_n, W_post, b_post):
    n, d = node_fts.shape
    grid = (n // N_BLOCK,)
    blk = lambda i: (i, 0)
    return pl.pallas_call(
        _post_kernel,
        grid=grid,
        in_specs=[
            pl.BlockSpec((N_BLOCK, d), blk),
            pl.BlockSpec((N_BLOCK, d), blk),
            pl.BlockSpec((N_BLOCK, d), blk),
            pl.BlockSpec((N_BLOCK, d), blk),
            pl.BlockSpec((N_BLOCK, 1), blk),
            pl.BlockSpec((N_BLOCK, 1), blk),
            pl.BlockSpec((N_BLOCK, 1), blk),
            pl.BlockSpec((4 * d, d), lambda i: (0, 0)),
            pl.BlockSpec((1, d), lambda i: (0, 0)),
        ],
        out_specs=pl.BlockSpec((N_BLOCK, d), blk),
        out_shape=jax.ShapeDtypeStruct((n, d), jnp.float32),
    )(node_fts, s, m, dirsum, deg, fdig, norm_n, W_post, b_post)


def kernel(node_fts, edge_fts, edge_index, F_norm_edge, F_dig, node_deg_vec,
           node_deg_mat, lap_mat, k_eig_val, k_eig_vec, num_nodes, norm_n,
           batch_idx, W_post, b_post):
    src = edge_index[0]
    dst = edge_index[1]
    f = F_norm_edge[:, 0]
    nf2 = node_fts.reshape(2 * N, DH)
    s3, m3, dir3 = _agg(nf2, src, dst, f)
    s = jnp.concatenate([s3[0, :N], s3[1, :N]], axis=1)
    m = jnp.concatenate([m3[0, :N], m3[1, :N]], axis=1)
    dirsum = jnp.concatenate([dir3[0, :N], dir3[1, :N]], axis=1)
    return _post_transform(node_fts, s, m, dirsum, node_deg_vec, F_dig,
                           norm_n, W_post, b_post[None, :])
